# baseline (speedup 1.0000x reference)
import functools

import numpy as np
import jax
import jax.numpy as jnp
from jax import lax
from jax.experimental import pallas as pl
from jax.experimental.pallas import tpu as pltpu
from jax.experimental.pallas import tpu_sc as plsc

_L = 16
_F = 2
_T = 2 ** 19
_NPTS = 65536
_B = float(np.exp(np.log(4096.0 / 16.0) / (_L - 1)))
_RES = [int(np.floor(16 * (_B ** l))) for l in range(_L)]
_P2I = int(np.uint32(2654435761).view(np.int32))
_P3I = int(np.uint32(805459861).view(np.int32))

_NW = 32
_PPW = _NPTS // _NW
_CPTS = 16
_NCH = _PPW // _CPTS
_D = 16
_OC = 112


_PR = 1024
_PNCH = (_T // 2) // _PR


def _sc_pack(t0, t1, t2, t3, t4):
    mesh = plsc.VectorSubcoreMesh(core_axis_name="c", subcore_axis_name="s")

    @functools.partial(
        pl.kernel,
        mesh=mesh,
        out_type=jax.ShapeDtypeStruct((_L * _T, _D), jnp.float32),
        compiler_params=pltpu.CompilerParams(
            needs_layout_passes=False, use_tc_tiling_on_sc=False),
        scratch_types=[
            pltpu.VMEM((2, 5, _PR, _F), jnp.float32),
            pltpu.VMEM((2, _PR, _D), jnp.float32),
            pltpu.SemaphoreType.DMA,
            pltpu.SemaphoreType.DMA,
            pltpu.SemaphoreType.DMA,
            pltpu.SemaphoreType.DMA,
        ],
    )
    def k(t0_h, t1_h, t2_h, t3_h, t4_h, out_h, in_s, out_s,
          semi0, semi1, semo0, semo1):
        tabs = (t0_h, t1_h, t2_h, t3_h, t4_h)
        semi = (semi0, semi1)
        semo = (semo0, semo1)
        wid = lax.axis_index("s") * 2 + lax.axis_index("c")
        lvl = wid // 2
        h0 = (wid % 2) * (_T // 2)
        it = lax.iota(jnp.int32, 16)
        z16 = it * 0
        rhalf = it // 2
        lane01 = it - rhalf * 2
        zero16 = z16.astype(jnp.float32)

        def zrow(r, c0):
            out_s[0, r, :] = zero16
            out_s[1, r, :] = zero16
            return c0

        lax.fori_loop(0, _PR, zrow, 0)

        def fire_in(kc, buf):
            row0 = h0 + kc * _PR
            return [
                pltpu.async_copy(tabs[t].at[lvl, pl.ds(row0, _PR), :],
                                 in_s.at[buf, t], semi[buf])
                for t in range(5)
            ]

        def drain_in(kc, buf):
            row0 = h0 + kc * _PR
            for t in range(5):
                pltpu.make_async_copy(tabs[t].at[lvl, pl.ds(row0, _PR), :],
                                      in_s.at[buf, t], semi[buf]).wait()

        def out_slice(kc):
            return out_h.at[pl.ds((lvl * _T + h0 + kc * _PR), _PR), :]

        def interleave(kc, buf):
            bvec = z16 + buf

            def gbody(g, c0):
                rv = g * 8 + rhalf
                for t in range(5):
                    tvec = z16 + t
                    chv = 2 * t + lane01
                    v = plsc.load_gather(in_s, [bvec, tvec, rv, lane01])
                    plsc.store_scatter(out_s, [bvec, rv, chv], v)
                return c0

            lax.fori_loop(0, _PR * _F // 16, gbody, 0)

        def step(kc, buf, first):
            nxt = jnp.minimum(kc + 1, _PNCH - 1)
            fire_in(nxt, 1 - buf)
            drain_in(kc, buf)
            if not first:
                pltpu.make_async_copy(out_s.at[buf],
                                      out_slice(jnp.maximum(kc - 2, 0)),
                                      semo[buf]).wait()
            interleave(kc, buf)
            pltpu.async_copy(out_s.at[buf], out_slice(kc), semo[buf])

        fire_in(0, 0)
        step(0, 0, True)
        step(1, 1, True)

        def body(kc2, c0):
            step(kc2 * 2, 0, False)
            step(kc2 * 2 + 1, 1, False)
            return c0

        lax.fori_loop(1, _PNCH // 2, body, 0)
        drain_in(_PNCH - 1, 0)
        pltpu.make_async_copy(out_s.at[0], out_slice(_PNCH - 2), semo[0]).wait()
        pltpu.make_async_copy(out_s.at[1], out_slice(_PNCH - 1), semo[1]).wait()

    return k(t0, t1, t2, t3, t4)


def _sc_encode(xs, ys, zs, tbl, resv, parv):
    mesh = plsc.VectorSubcoreMesh(core_axis_name="c", subcore_axis_name="s")

    @functools.partial(
        pl.kernel,
        mesh=mesh,
        out_type=jax.ShapeDtypeStruct((_OC, _NPTS), jnp.float32),
        compiler_params=pltpu.CompilerParams(
            needs_layout_passes=False, use_tc_tiling_on_sc=False),
        scratch_types=[
            pltpu.VMEM((_PPW,), jnp.float32),
            pltpu.VMEM((_PPW,), jnp.float32),
            pltpu.VMEM((_PPW,), jnp.float32),
            pltpu.VMEM((16,), jnp.float32),
            pltpu.VMEM((16,), jnp.float32),
            pltpu.VMEM((2, _L, 8 * _CPTS), jnp.int32),
            pltpu.VMEM((2, _L, 8 * _CPTS), jnp.float32),
            pltpu.VMEM((2, _L, 8 * _CPTS, _D), jnp.float32),
            pltpu.VMEM((_OC, _CPTS), jnp.float32),
            pltpu.SemaphoreType.DMA,
            pltpu.SemaphoreType.DMA,
        ],
    )
    def k(xs_h, ys_h, zs_h, tbl_h, res_h, par_h, out_h,
          x_s, y_s, z_s, res_s, par_s, idx_s, w_s, rows_s, stage_s, sem0, sem1):
        wid = lax.axis_index("s") * 2 + lax.axis_index("c")
        base = wid * _PPW
        pltpu.sync_copy(xs_h.at[pl.ds(base, _PPW)], x_s)
        pltpu.sync_copy(ys_h.at[pl.ds(base, _PPW)], y_s)
        pltpu.sync_copy(zs_h.at[pl.ds(base, _PPW)], z_s)
        pltpu.sync_copy(res_h, res_s)
        pltpu.sync_copy(par_h, par_s)
        it = lax.iota(jnp.int32, 16)
        z16 = it * 0
        a1 = plsc.load_gather(par_s, [z16])
        b1 = plsc.load_gather(par_s, [z16 + 1])
        a2 = plsc.load_gather(par_s, [z16 + 2])
        b2 = plsc.load_gather(par_s, [z16 + 3])
        sems = (sem0, sem1)

        def compute_indices(kc, buf):
            po = kc * _CPTS
            x = x_s[pl.ds(po, _CPTS)] * 0.5 + 0.5
            y = y_s[pl.ds(po, _CPTS)] * 0.5 + 0.5
            z = z_s[pl.ds(po, _CPTS)] * 0.5 + 0.5

            def lvl_idx(l, c2):
                r = plsc.load_gather(res_s, [z16 + l])
                px = x * r
                py = y * r
                pz = z * r
                ix = px.astype(jnp.int32)
                iy = py.astype(jnp.int32)
                iz = pz.astype(jnp.int32)
                fx = px - ix.astype(jnp.float32)
                fy = py - iy.astype(jnp.float32)
                fz = pz - iz.astype(jnp.float32)
                gy = iy * _P2I
                gz = iz * _P3I
                lT = l * _T
                for c in range(8):
                    cx, cy, cz = c & 1, (c >> 1) & 1, (c >> 2) & 1
                    hx = ix + cx if cx else ix
                    hy = gy + _P2I if cy else gy
                    hz = gz + _P3I if cz else gz
                    h = ((hx ^ hy) ^ hz) & (_T - 1)
                    wx = fx if cx else 1.0 - fx
                    wy = fy if cy else 1.0 - fy
                    wz = fz if cz else 1.0 - fz
                    idx_s[buf, l, pl.ds(c * _CPTS, _CPTS)] = h + lT
                    w_s[buf, l, pl.ds(c * _CPTS, _CPTS)] = wx * wy * wz
                return c2

            lax.fori_loop(0, _L, lvl_idx, 0)

        def fire(buf):
            return [
                pltpu.async_copy(tbl_h.at[idx_s.at[buf, i]], rows_s.at[buf, i],
                                 sems[buf])
                for i in range(_L)
            ]

        def accumulate(kc, buf):
            po = kc * _CPTS

            def lvl_acc(l, c2):
                bvec = z16 + buf
                lvec = z16 + l
                acc = [jnp.zeros((16,), jnp.float32) for _ in range(10)]
                for c in range(8):
                    w = w_s[buf, l, pl.ds(c * _CPTS, _CPTS)]
                    rvec = c * _CPTS + it
                    for j in range(10):
                        cvec = z16 + j
                        v = plsc.load_gather(rows_s, [bvec, lvec, rvec, cvec])
                        acc[j] = acc[j] + w * v
                col = 2 * l
                plsc.store_scatter(stage_s, [z16 + col, it], acc[0])
                plsc.store_scatter(stage_s, [z16 + (col + 1), it], acc[1])
                plsc.store_scatter(stage_s, [z16 + (32 + col), it], a1 * acc[2] + b1 * acc[4])
                plsc.store_scatter(stage_s, [z16 + (33 + col), it], a1 * acc[3] + b1 * acc[5])
                plsc.store_scatter(stage_s, [z16 + (64 + col), it], a2 * acc[6] + b2 * acc[8])
                plsc.store_scatter(stage_s, [z16 + (65 + col), it], a2 * acc[7] + b2 * acc[9])

                @pl.when(l >= 12)
                def _():
                    colA = 96 + 2 * (l - 12)
                    plsc.store_scatter(stage_s, [z16 + colA, it], acc[2])
                    plsc.store_scatter(stage_s, [z16 + (colA + 1), it], acc[3])
                    plsc.store_scatter(stage_s, [z16 + (colA + 8), it], acc[4])
                    plsc.store_scatter(stage_s, [z16 + (colA + 9), it], acc[5])

                return c2

            lax.fori_loop(0, _L, lvl_acc, 0)
            pltpu.sync_copy(stage_s, out_h.at[:, pl.ds(base + po, _CPTS)])

        compute_indices(0, 0)
        fire(0)

        def step(kc, buf):
            nxt = jnp.minimum(kc + 1, _NCH - 1)
            compute_indices(nxt, 1 - buf)
            fire(1 - buf)
            for i in range(_L):
                pltpu.make_async_copy(
                    tbl_h.at[idx_s.at[buf, i]], rows_s.at[buf, i], sems[buf]
                ).wait()
            accumulate(kc, buf)

        def body(kc2, c0):
            step(kc2 * 2, 0)
            step(kc2 * 2 + 1, 1)
            return c0

        lax.fori_loop(0, _NCH // 2, body, 0)
        for i in range(_L):
            pltpu.make_async_copy(
                tbl_h.at[idx_s.at[0, i]], rows_s.at[0, i], sems[0]
            ).wait()

    return k(xs, ys, zs, tbl, resv, parv)


_BP = 2048


def _mlp_body(sc_ref, dirt_ref, w1t_ref, w2t_ref, c1t_ref, c2t_ref, c3t_ref,
              sig_ref, col_ref):
    featt = sc_ref[:96, :]
    h1t = jnp.maximum(jnp.dot(w1t_ref[...], featt,
                              preferred_element_type=jnp.float32), 0.0)
    ht = jnp.dot(w2t_ref[...], h1t, preferred_element_type=jnp.float32)
    sig_ref[...] = jnp.exp(ht[0:1, :])

    d = dirt_ref[...]
    x = d[0:1, :]
    y = d[1:2, :]
    z = d[2:3, :]
    inv = 1.0 / (jnp.sqrt(x * x + y * y + z * z) + 1e-8)
    x = x * inv
    y = y * inv
    z = z * inv
    x2, y2, z2 = x * x, y * y, z * z
    xy, yz, xz = x * y, y * z, x * z
    comps = [
        0.28209479177387814 * jnp.ones_like(x),
        -0.48860251190291987 * y,
        0.48860251190291987 * z,
        -0.48860251190291987 * x,
        1.0925484305920792 * xy,
        -1.0925484305920792 * yz,
        0.94617469575755997 * z2 - 0.31539156525252005,
        -1.0925484305920792 * xz,
        0.54627421529603959 * (x2 - y2),
        -0.59004358992664352 * y * (3.0 * x2 - y2),
        2.8906114426405538 * xy * z,
        -0.45704579946446572 * y * (4.0 * z2 - x2 - y2),
        0.3731763325901154 * z * (2.0 * z2 - 3.0 * x2 - 3.0 * y2),
        -0.45704579946446572 * x * (4.0 * z2 - x2 - y2),
        1.4453057213202769 * z * (x2 - y2),
        -0.59004358992664352 * x * (x2 - 3.0 * y2),
    ]
    sht = jnp.concatenate(comps, axis=0)
    ci1 = (jnp.dot(c1t_ref[:, :16], sht, preferred_element_type=jnp.float32)
           + jnp.dot(c1t_ref[:, 16:], ht, preferred_element_type=jnp.float32))
    cc = jnp.maximum(ci1, 0.0)
    cc = jnp.maximum(jnp.dot(c2t_ref[...], cc, preferred_element_type=jnp.float32), 0.0)
    col_ref[...] = jax.nn.sigmoid(
        jnp.dot(c3t_ref[...], cc, preferred_element_type=jnp.float32))


def _tc_mlp(sc_out, dirt, W1t, W2t, C1t, C2t, C3t):
    grid = (_NPTS // _BP,)
    return pl.pallas_call(
        _mlp_body,
        grid=grid,
        in_specs=[
            pl.BlockSpec((_OC, _BP), lambda i: (0, i)),
            pl.BlockSpec((3, _BP), lambda i: (0, i)),
            pl.BlockSpec((64, 96), lambda i: (0, 0)),
            pl.BlockSpec((16, 64), lambda i: (0, 0)),
            pl.BlockSpec((64, 32), lambda i: (0, 0)),
            pl.BlockSpec((64, 64), lambda i: (0, 0)),
            pl.BlockSpec((3, 64), lambda i: (0, 0)),
        ],
        out_specs=[
            pl.BlockSpec((1, _BP), lambda i: (0, i)),
            pl.BlockSpec((3, _BP), lambda i: (0, i)),
        ],
        out_shape=[
            jax.ShapeDtypeStruct((1, _NPTS), jnp.float32),
            jax.ShapeDtypeStruct((3, _NPTS), jnp.float32),
        ],
    )(sc_out, dirt, W1t, W2t, C1t, C2t, C3t)


def kernel(original_xyzs, dirs, static_table, tableA, tableB, table2A, table2B,
           W1, W2, C1, C2, C3):
    xs = original_xyzs[:, 0]
    ys = original_xyzs[:, 1]
    zs = original_xyzs[:, 2]
    t0 = original_xyzs[0, 3]

    prev1 = 1.0 - (t0 * 16.0 - 8.0)
    nxt1 = 1.0 - prev1
    s1 = prev1 + nxt1
    prev2 = 1.0 - (t0 * 20.0 - 10.0)
    nxt2 = 1.0 - prev2
    s2 = prev2 + nxt2
    par = jnp.concatenate([
        jnp.stack([prev1 / s1, nxt1 / s1, prev2 / s2, nxt2 / s2]),
        jnp.zeros((12,), jnp.float32),
    ])
    resv = jnp.asarray(_RES, dtype=jnp.float32)

    tbl = _sc_pack(static_table, tableA, tableB, table2A, table2B)

    sc_out = _sc_encode(xs, ys, zs, tbl, resv, par)

    sigt, colt = _tc_mlp(sc_out, dirs.T, W1.T, W2.T, C1.T, C2.T, C3.T)
    sigma = sigt.reshape(_NPTS)
    color = colt.T
    af1 = sc_out[96:104, :].T
    af2 = sc_out[104:112, :].T
    return (sigma, color, af1, af2)


# speedup vs baseline: 1.2361x; 1.2361x over previous
import functools

import numpy as np
import jax
import jax.numpy as jnp
from jax import lax
from jax.experimental import pallas as pl
from jax.experimental.pallas import tpu as pltpu
from jax.experimental.pallas import tpu_sc as plsc

_L = 16
_F = 2
_T = 2 ** 19
_NPTS = 65536
_B = float(np.exp(np.log(4096.0 / 16.0) / (_L - 1)))
_RES = [int(np.floor(16 * (_B ** l))) for l in range(_L)]
_P2I = int(np.uint32(2654435761).view(np.int32))
_P3I = int(np.uint32(805459861).view(np.int32))

_NW = 32
_PPW = _NPTS // _NW
_CPTS = 16
_NCH = _PPW // _CPTS
_D = 16
_OC = 112


_PR = 1024
_PNCH = (_T // 2) // _PR


def _sc_pack(t0, t1, t2, t3, t4):
    mesh = plsc.VectorSubcoreMesh(core_axis_name="c", subcore_axis_name="s")

    @functools.partial(
        pl.kernel,
        mesh=mesh,
        out_type=jax.ShapeDtypeStruct((_L * _T, _D), jnp.float32),
        compiler_params=pltpu.CompilerParams(
            needs_layout_passes=False, use_tc_tiling_on_sc=False),
        scratch_types=[
            pltpu.VMEM((2, 5, _PR * _F), jnp.float32),
            pltpu.VMEM((2, _PR, _D), jnp.float32),
            pltpu.SemaphoreType.DMA,
            pltpu.SemaphoreType.DMA,
            pltpu.SemaphoreType.DMA,
            pltpu.SemaphoreType.DMA,
        ],
    )
    def k(t0_h, t1_h, t2_h, t3_h, t4_h, out_h, in_s, out_s,
          semi0, semi1, semo0, semo1):
        tabs = (t0_h, t1_h, t2_h, t3_h, t4_h)
        semi = (semi0, semi1)
        semo = (semo0, semo1)
        wid = lax.axis_index("s") * 2 + lax.axis_index("c")
        lvl = wid // 2
        h0 = (wid % 2) * (_T // 2)
        it = lax.iota(jnp.int32, 16)
        z16 = it * 0
        rhalf = it // 2
        lane01 = it - rhalf * 2
        zero16 = z16.astype(jnp.float32)

        def zrow(r, c0):
            out_s[0, r, :] = zero16
            out_s[1, r, :] = zero16
            return c0

        lax.fori_loop(0, _PR, zrow, 0)

        def fire_in(kc, buf):
            e0 = (lvl * _T + h0 + kc * _PR) * _F
            return [
                pltpu.async_copy(tabs[t].at[pl.ds(e0, _PR * _F)],
                                 in_s.at[buf, t], semi[buf])
                for t in range(5)
            ]

        def drain_in(kc, buf):
            e0 = (lvl * _T + h0 + kc * _PR) * _F
            for t in range(5):
                pltpu.make_async_copy(tabs[t].at[pl.ds(e0, _PR * _F)],
                                      in_s.at[buf, t], semi[buf]).wait()

        def out_slice(kc):
            return out_h.at[pl.ds((lvl * _T + h0 + kc * _PR), _PR), :]

        def interleave(kc, buf):
            bvec = z16 + buf

            def gbody(g, c0):
                rv = g * 8 + rhalf
                for t in range(5):
                    chv = 2 * t + lane01
                    v = plsc.load_gather(in_s, [bvec, z16 + t, g * 16 + it])
                    plsc.store_scatter(out_s, [bvec, rv, chv], v)
                return c0

            lax.fori_loop(0, _PR * _F // 16, gbody, 0)

        def step(kc, buf, first):
            nxt = jnp.minimum(kc + 1, _PNCH - 1)
            fire_in(nxt, 1 - buf)
            drain_in(kc, buf)
            if not first:
                pltpu.make_async_copy(out_s.at[buf],
                                      out_slice(jnp.maximum(kc - 2, 0)),
                                      semo[buf]).wait()
            interleave(kc, buf)
            pltpu.async_copy(out_s.at[buf], out_slice(kc), semo[buf])

        fire_in(0, 0)
        step(0, 0, True)
        step(1, 1, True)

        def body(kc2, c0):
            step(kc2 * 2, 0, False)
            step(kc2 * 2 + 1, 1, False)
            return c0

        lax.fori_loop(1, _PNCH // 2, body, 0)
        drain_in(_PNCH - 1, 0)
        pltpu.make_async_copy(out_s.at[0], out_slice(_PNCH - 2), semo[0]).wait()
        pltpu.make_async_copy(out_s.at[1], out_slice(_PNCH - 1), semo[1]).wait()

    return k(t0, t1, t2, t3, t4)


def _sc_encode(xs, ys, zs, tbl, resv, parv):
    mesh = plsc.VectorSubcoreMesh(core_axis_name="c", subcore_axis_name="s")

    @functools.partial(
        pl.kernel,
        mesh=mesh,
        out_type=jax.ShapeDtypeStruct((_OC, _NPTS), jnp.float32),
        compiler_params=pltpu.CompilerParams(
            needs_layout_passes=False, use_tc_tiling_on_sc=False),
        scratch_types=[
            pltpu.VMEM((_PPW,), jnp.float32),
            pltpu.VMEM((_PPW,), jnp.float32),
            pltpu.VMEM((_PPW,), jnp.float32),
            pltpu.VMEM((16,), jnp.float32),
            pltpu.VMEM((16,), jnp.float32),
            pltpu.VMEM((2, _L, 8 * _CPTS), jnp.int32),
            pltpu.VMEM((2, _L, 8 * _CPTS), jnp.float32),
            pltpu.VMEM((2, _L, 8 * _CPTS, _D), jnp.float32),
            pltpu.VMEM((_OC, _CPTS), jnp.float32),
            pltpu.SemaphoreType.DMA,
            pltpu.SemaphoreType.DMA,
        ],
    )
    def k(xs_h, ys_h, zs_h, tbl_h, res_h, par_h, out_h,
          x_s, y_s, z_s, res_s, par_s, idx_s, w_s, rows_s, stage_s, sem0, sem1):
        wid = lax.axis_index("s") * 2 + lax.axis_index("c")
        base = wid * _PPW
        pltpu.sync_copy(xs_h.at[pl.ds(base, _PPW)], x_s)
        pltpu.sync_copy(ys_h.at[pl.ds(base, _PPW)], y_s)
        pltpu.sync_copy(zs_h.at[pl.ds(base, _PPW)], z_s)
        pltpu.sync_copy(res_h, res_s)
        pltpu.sync_copy(par_h, par_s)
        it = lax.iota(jnp.int32, 16)
        z16 = it * 0
        a1 = plsc.load_gather(par_s, [z16])
        b1 = plsc.load_gather(par_s, [z16 + 1])
        a2 = plsc.load_gather(par_s, [z16 + 2])
        b2 = plsc.load_gather(par_s, [z16 + 3])
        sems = (sem0, sem1)

        def compute_indices(kc, buf):
            po = kc * _CPTS
            x = x_s[pl.ds(po, _CPTS)] * 0.5 + 0.5
            y = y_s[pl.ds(po, _CPTS)] * 0.5 + 0.5
            z = z_s[pl.ds(po, _CPTS)] * 0.5 + 0.5

            def lvl_idx(l, c2):
                r = plsc.load_gather(res_s, [z16 + l])
                px = x * r
                py = y * r
                pz = z * r
                ix = px.astype(jnp.int32)
                iy = py.astype(jnp.int32)
                iz = pz.astype(jnp.int32)
                fx = px - ix.astype(jnp.float32)
                fy = py - iy.astype(jnp.float32)
                fz = pz - iz.astype(jnp.float32)
                gy = iy * _P2I
                gz = iz * _P3I
                lT = l * _T
                for c in range(8):
                    cx, cy, cz = c & 1, (c >> 1) & 1, (c >> 2) & 1
                    hx = ix + cx if cx else ix
                    hy = gy + _P2I if cy else gy
                    hz = gz + _P3I if cz else gz
                    h = ((hx ^ hy) ^ hz) & (_T - 1)
                    wx = fx if cx else 1.0 - fx
                    wy = fy if cy else 1.0 - fy
                    wz = fz if cz else 1.0 - fz
                    idx_s[buf, l, pl.ds(c * _CPTS, _CPTS)] = h + lT
                    w_s[buf, l, pl.ds(c * _CPTS, _CPTS)] = wx * wy * wz
                return c2

            lax.fori_loop(0, _L, lvl_idx, 0)

        def fire(buf):
            return [
                pltpu.async_copy(tbl_h.at[idx_s.at[buf, i]], rows_s.at[buf, i],
                                 sems[buf])
                for i in range(_L)
            ]

        def accumulate(kc, buf):
            po = kc * _CPTS

            def lvl_acc(l, c2):
                bvec = z16 + buf
                lvec = z16 + l
                acc = [jnp.zeros((16,), jnp.float32) for _ in range(10)]
                for c in range(8):
                    w = w_s[buf, l, pl.ds(c * _CPTS, _CPTS)]
                    rvec = c * _CPTS + it
                    for j in range(10):
                        cvec = z16 + j
                        v = plsc.load_gather(rows_s, [bvec, lvec, rvec, cvec])
                        acc[j] = acc[j] + w * v
                col = 2 * l
                plsc.store_scatter(stage_s, [z16 + col, it], acc[0])
                plsc.store_scatter(stage_s, [z16 + (col + 1), it], acc[1])
                plsc.store_scatter(stage_s, [z16 + (32 + col), it], a1 * acc[2] + b1 * acc[4])
                plsc.store_scatter(stage_s, [z16 + (33 + col), it], a1 * acc[3] + b1 * acc[5])
                plsc.store_scatter(stage_s, [z16 + (64 + col), it], a2 * acc[6] + b2 * acc[8])
                plsc.store_scatter(stage_s, [z16 + (65 + col), it], a2 * acc[7] + b2 * acc[9])

                @pl.when(l >= 12)
                def _():
                    colA = 96 + 2 * (l - 12)
                    plsc.store_scatter(stage_s, [z16 + colA, it], acc[2])
                    plsc.store_scatter(stage_s, [z16 + (colA + 1), it], acc[3])
                    plsc.store_scatter(stage_s, [z16 + (colA + 8), it], acc[4])
                    plsc.store_scatter(stage_s, [z16 + (colA + 9), it], acc[5])

                return c2

            lax.fori_loop(0, _L, lvl_acc, 0)
            pltpu.sync_copy(stage_s, out_h.at[:, pl.ds(base + po, _CPTS)])

        compute_indices(0, 0)
        fire(0)

        def step(kc, buf):
            nxt = jnp.minimum(kc + 1, _NCH - 1)
            compute_indices(nxt, 1 - buf)
            fire(1 - buf)
            for i in range(_L):
                pltpu.make_async_copy(
                    tbl_h.at[idx_s.at[buf, i]], rows_s.at[buf, i], sems[buf]
                ).wait()
            accumulate(kc, buf)

        def body(kc2, c0):
            step(kc2 * 2, 0)
            step(kc2 * 2 + 1, 1)
            return c0

        lax.fori_loop(0, _NCH // 2, body, 0)
        for i in range(_L):
            pltpu.make_async_copy(
                tbl_h.at[idx_s.at[0, i]], rows_s.at[0, i], sems[0]
            ).wait()

    return k(xs, ys, zs, tbl, resv, parv)


_BP = 2048


def _mlp_body(sc_ref, dirt_ref, w1t_ref, w2t_ref, c1t_ref, c2t_ref, c3t_ref,
              sig_ref, col_ref):
    featt = sc_ref[:96, :]
    h1t = jnp.maximum(jnp.dot(w1t_ref[...], featt,
                              preferred_element_type=jnp.float32), 0.0)
    ht = jnp.dot(w2t_ref[...], h1t, preferred_element_type=jnp.float32)
    sig_ref[...] = jnp.exp(ht[0:1, :])

    d = dirt_ref[...]
    x = d[0:1, :]
    y = d[1:2, :]
    z = d[2:3, :]
    inv = 1.0 / (jnp.sqrt(x * x + y * y + z * z) + 1e-8)
    x = x * inv
    y = y * inv
    z = z * inv
    x2, y2, z2 = x * x, y * y, z * z
    xy, yz, xz = x * y, y * z, x * z
    comps = [
        0.28209479177387814 * jnp.ones_like(x),
        -0.48860251190291987 * y,
        0.48860251190291987 * z,
        -0.48860251190291987 * x,
        1.0925484305920792 * xy,
        -1.0925484305920792 * yz,
        0.94617469575755997 * z2 - 0.31539156525252005,
        -1.0925484305920792 * xz,
        0.54627421529603959 * (x2 - y2),
        -0.59004358992664352 * y * (3.0 * x2 - y2),
        2.8906114426405538 * xy * z,
        -0.45704579946446572 * y * (4.0 * z2 - x2 - y2),
        0.3731763325901154 * z * (2.0 * z2 - 3.0 * x2 - 3.0 * y2),
        -0.45704579946446572 * x * (4.0 * z2 - x2 - y2),
        1.4453057213202769 * z * (x2 - y2),
        -0.59004358992664352 * x * (x2 - 3.0 * y2),
    ]
    sht = jnp.concatenate(comps, axis=0)
    ci1 = (jnp.dot(c1t_ref[:, :16], sht, preferred_element_type=jnp.float32)
           + jnp.dot(c1t_ref[:, 16:], ht, preferred_element_type=jnp.float32))
    cc = jnp.maximum(ci1, 0.0)
    cc = jnp.maximum(jnp.dot(c2t_ref[...], cc, preferred_element_type=jnp.float32), 0.0)
    col_ref[...] = jax.nn.sigmoid(
        jnp.dot(c3t_ref[...], cc, preferred_element_type=jnp.float32))


def _tc_mlp(sc_out, dirt, W1t, W2t, C1t, C2t, C3t):
    grid = (_NPTS // _BP,)
    return pl.pallas_call(
        _mlp_body,
        grid=grid,
        in_specs=[
            pl.BlockSpec((_OC, _BP), lambda i: (0, i)),
            pl.BlockSpec((3, _BP), lambda i: (0, i)),
            pl.BlockSpec((64, 96), lambda i: (0, 0)),
            pl.BlockSpec((16, 64), lambda i: (0, 0)),
            pl.BlockSpec((64, 32), lambda i: (0, 0)),
            pl.BlockSpec((64, 64), lambda i: (0, 0)),
            pl.BlockSpec((3, 64), lambda i: (0, 0)),
        ],
        out_specs=[
            pl.BlockSpec((1, _BP), lambda i: (0, i)),
            pl.BlockSpec((3, _BP), lambda i: (0, i)),
        ],
        out_shape=[
            jax.ShapeDtypeStruct((1, _NPTS), jnp.float32),
            jax.ShapeDtypeStruct((3, _NPTS), jnp.float32),
        ],
    )(sc_out, dirt, W1t, W2t, C1t, C2t, C3t)


def kernel(original_xyzs, dirs, static_table, tableA, tableB, table2A, table2B,
           W1, W2, C1, C2, C3):
    xs = original_xyzs[:, 0]
    ys = original_xyzs[:, 1]
    zs = original_xyzs[:, 2]
    t0 = original_xyzs[0, 3]

    prev1 = 1.0 - (t0 * 16.0 - 8.0)
    nxt1 = 1.0 - prev1
    s1 = prev1 + nxt1
    prev2 = 1.0 - (t0 * 20.0 - 10.0)
    nxt2 = 1.0 - prev2
    s2 = prev2 + nxt2
    par = jnp.concatenate([
        jnp.stack([prev1 / s1, nxt1 / s1, prev2 / s2, nxt2 / s2]),
        jnp.zeros((12,), jnp.float32),
    ])
    resv = jnp.asarray(_RES, dtype=jnp.float32)

    tbl = _sc_pack(static_table.reshape(-1), tableA.reshape(-1),
                   tableB.reshape(-1), table2A.reshape(-1), table2B.reshape(-1))

    sc_out = _sc_encode(xs, ys, zs, tbl, resv, par)

    sigt, colt = _tc_mlp(sc_out, dirs.T, W1.T, W2.T, C1.T, C2.T, C3.T)
    sigma = sigt.reshape(_NPTS)
    color = colt.T
    af1 = sc_out[96:104, :].T
    af2 = sc_out[104:112, :].T
    return (sigma, color, af1, af2)


# speedup vs baseline: 43.0660x; 34.8409x over previous
import functools

import numpy as np
import jax
import jax.numpy as jnp
from jax import lax
from jax.experimental import pallas as pl
from jax.experimental.pallas import tpu as pltpu
from jax.experimental.pallas import tpu_sc as plsc

_L = 16
_F = 2
_T = 2 ** 19
_NPTS = 65536
_B = float(np.exp(np.log(4096.0 / 16.0) / (_L - 1)))
_RES = [int(np.floor(16 * (_B ** l))) for l in range(_L)]
_P2I = int(np.uint32(2654435761).view(np.int32))
_P3I = int(np.uint32(805459861).view(np.int32))

_NW = 32
_PPW = _NPTS // _NW
_CPTS = 16
_NCH = _PPW // _CPTS
_D = 16
_OC = 112


_PR = 1024
_PNCH = (_T // 2) // _PR


def _sc_pack(t0, t1, t2, t3, t4):
    mesh = plsc.VectorSubcoreMesh(core_axis_name="c", subcore_axis_name="s")

    @functools.partial(
        pl.kernel,
        mesh=mesh,
        out_type=jax.ShapeDtypeStruct((_L * _T, _D), jnp.float32),
        compiler_params=pltpu.CompilerParams(
            needs_layout_passes=False, use_tc_tiling_on_sc=False),
        scratch_types=[
            pltpu.VMEM((2, 5, _PR * _F), jnp.float32),
            pltpu.VMEM((2, _PR, _D), jnp.float32),
            pltpu.SemaphoreType.DMA,
            pltpu.SemaphoreType.DMA,
            pltpu.SemaphoreType.DMA,
            pltpu.SemaphoreType.DMA,
        ],
    )
    def k(t0_h, t1_h, t2_h, t3_h, t4_h, out_h, in_s, out_s,
          semi0, semi1, semo0, semo1):
        tabs = (t0_h, t1_h, t2_h, t3_h, t4_h)
        semi = (semi0, semi1)
        semo = (semo0, semo1)
        wid = lax.axis_index("s") * 2 + lax.axis_index("c")
        lvl = wid // 2
        h0 = (wid % 2) * (_T // 2)
        it = lax.iota(jnp.int32, 16)
        z16 = it * 0
        rhalf = it // 2
        lane01 = it - rhalf * 2
        zero16 = z16.astype(jnp.float32)

        def zrow(r, c0):
            out_s[0, r, :] = zero16
            out_s[1, r, :] = zero16
            return c0

        lax.fori_loop(0, _PR, zrow, 0)

        def fire_in(kc, buf):
            e0 = lvl * (_T * _F) + ((h0 + kc * _PR) // 128) * 256
            return [
                pltpu.async_copy(tabs[t].at[pl.ds(e0, _PR * _F)],
                                 in_s.at[buf, t], semi[buf])
                for t in range(5)
            ]

        def drain_in(kc, buf):
            e0 = lvl * (_T * _F) + ((h0 + kc * _PR) // 128) * 256
            for t in range(5):
                pltpu.make_async_copy(tabs[t].at[pl.ds(e0, _PR * _F)],
                                      in_s.at[buf, t], semi[buf]).wait()

        def out_slice(kc):
            return out_h.at[pl.ds((lvl * _T + h0 + kc * _PR), _PR), :]

        def interleave(kc, buf):
            bvec = z16 + buf

            def gbody(g, c0):
                blk = g // 16
                c = (g // 8) - blk * 2
                rr = g - blk * 16 - c * 8
                rv = blk * 128 + rr * 16 + it
                src = blk * 256 + c * 128 + rr * 16 + it
                for t in range(5):
                    chv = z16 + (2 * t + c)
                    v = plsc.load_gather(in_s, [bvec, z16 + t, src])
                    plsc.store_scatter(out_s, [bvec, rv, chv], v)
                return c0

            lax.fori_loop(0, _PR * _F // 16, gbody, 0)

        def step(kc, buf, first):
            nxt = jnp.minimum(kc + 1, _PNCH - 1)
            fire_in(nxt, 1 - buf)
            drain_in(kc, buf)
            if not first:
                pltpu.make_async_copy(out_s.at[buf],
                                      out_slice(jnp.maximum(kc - 2, 0)),
                                      semo[buf]).wait()
            interleave(kc, buf)
            pltpu.async_copy(out_s.at[buf], out_slice(kc), semo[buf])

        fire_in(0, 0)
        step(0, 0, True)
        step(1, 1, True)

        def body(kc2, c0):
            step(kc2 * 2, 0, False)
            step(kc2 * 2 + 1, 1, False)
            return c0

        lax.fori_loop(1, _PNCH // 2, body, 0)
        drain_in(_PNCH - 1, 0)
        pltpu.make_async_copy(out_s.at[0], out_slice(_PNCH - 2), semo[0]).wait()
        pltpu.make_async_copy(out_s.at[1], out_slice(_PNCH - 1), semo[1]).wait()

    return k(t0, t1, t2, t3, t4)


def _sc_encode(xs, ys, zs, tbl, resv, parv):
    mesh = plsc.VectorSubcoreMesh(core_axis_name="c", subcore_axis_name="s")

    @functools.partial(
        pl.kernel,
        mesh=mesh,
        out_type=jax.ShapeDtypeStruct((_OC, _NPTS), jnp.float32),
        compiler_params=pltpu.CompilerParams(
            needs_layout_passes=False, use_tc_tiling_on_sc=False),
        scratch_types=[
            pltpu.VMEM((_PPW,), jnp.float32),
            pltpu.VMEM((_PPW,), jnp.float32),
            pltpu.VMEM((_PPW,), jnp.float32),
            pltpu.VMEM((16,), jnp.float32),
            pltpu.VMEM((16,), jnp.float32),
            pltpu.VMEM((2, _L, 8 * _CPTS), jnp.int32),
            pltpu.VMEM((2, _L, 8 * _CPTS), jnp.float32),
            pltpu.VMEM((2, _L, 8 * _CPTS, _D), jnp.float32),
            pltpu.VMEM((_OC, _CPTS), jnp.float32),
            pltpu.SemaphoreType.DMA,
            pltpu.SemaphoreType.DMA,
        ],
    )
    def k(xs_h, ys_h, zs_h, tbl_h, res_h, par_h, out_h,
          x_s, y_s, z_s, res_s, par_s, idx_s, w_s, rows_s, stage_s, sem0, sem1):
        wid = lax.axis_index("s") * 2 + lax.axis_index("c")
        base = wid * _PPW
        pltpu.sync_copy(xs_h.at[pl.ds(base, _PPW)], x_s)
        pltpu.sync_copy(ys_h.at[pl.ds(base, _PPW)], y_s)
        pltpu.sync_copy(zs_h.at[pl.ds(base, _PPW)], z_s)
        pltpu.sync_copy(res_h, res_s)
        pltpu.sync_copy(par_h, par_s)
        it = lax.iota(jnp.int32, 16)
        z16 = it * 0
        a1 = plsc.load_gather(par_s, [z16])
        b1 = plsc.load_gather(par_s, [z16 + 1])
        a2 = plsc.load_gather(par_s, [z16 + 2])
        b2 = plsc.load_gather(par_s, [z16 + 3])
        sems = (sem0, sem1)

        def compute_indices(kc, buf):
            po = kc * _CPTS
            x = x_s[pl.ds(po, _CPTS)] * 0.5 + 0.5
            y = y_s[pl.ds(po, _CPTS)] * 0.5 + 0.5
            z = z_s[pl.ds(po, _CPTS)] * 0.5 + 0.5

            def lvl_idx(l, c2):
                r = plsc.load_gather(res_s, [z16 + l])
                px = x * r
                py = y * r
                pz = z * r
                ix = px.astype(jnp.int32)
                iy = py.astype(jnp.int32)
                iz = pz.astype(jnp.int32)
                fx = px - ix.astype(jnp.float32)
                fy = py - iy.astype(jnp.float32)
                fz = pz - iz.astype(jnp.float32)
                gy = iy * _P2I
                gz = iz * _P3I
                lT = l * _T
                for c in range(8):
                    cx, cy, cz = c & 1, (c >> 1) & 1, (c >> 2) & 1
                    hx = ix + cx if cx else ix
                    hy = gy + _P2I if cy else gy
                    hz = gz + _P3I if cz else gz
                    h = ((hx ^ hy) ^ hz) & (_T - 1)
                    wx = fx if cx else 1.0 - fx
                    wy = fy if cy else 1.0 - fy
                    wz = fz if cz else 1.0 - fz
                    idx_s[buf, l, pl.ds(c * _CPTS, _CPTS)] = h + lT
                    w_s[buf, l, pl.ds(c * _CPTS, _CPTS)] = wx * wy * wz
                return c2

            lax.fori_loop(0, _L, lvl_idx, 0)

        def fire(buf):
            return [
                pltpu.async_copy(tbl_h.at[idx_s.at[buf, i]], rows_s.at[buf, i],
                                 sems[buf])
                for i in range(_L)
            ]

        def accumulate(kc, buf):
            po = kc * _CPTS

            def lvl_acc(l, c2):
                bvec = z16 + buf
                lvec = z16 + l
                acc = [jnp.zeros((16,), jnp.float32) for _ in range(10)]
                for c in range(8):
                    w = w_s[buf, l, pl.ds(c * _CPTS, _CPTS)]
                    rvec = c * _CPTS + it
                    for j in range(10):
                        cvec = z16 + j
                        v = plsc.load_gather(rows_s, [bvec, lvec, rvec, cvec])
                        acc[j] = acc[j] + w * v
                col = 2 * l
                plsc.store_scatter(stage_s, [z16 + col, it], acc[0])
                plsc.store_scatter(stage_s, [z16 + (col + 1), it], acc[1])
                plsc.store_scatter(stage_s, [z16 + (32 + col), it], a1 * acc[2] + b1 * acc[4])
                plsc.store_scatter(stage_s, [z16 + (33 + col), it], a1 * acc[3] + b1 * acc[5])
                plsc.store_scatter(stage_s, [z16 + (64 + col), it], a2 * acc[6] + b2 * acc[8])
                plsc.store_scatter(stage_s, [z16 + (65 + col), it], a2 * acc[7] + b2 * acc[9])

                @pl.when(l >= 12)
                def _():
                    colA = 96 + 2 * (l - 12)
                    plsc.store_scatter(stage_s, [z16 + colA, it], acc[2])
                    plsc.store_scatter(stage_s, [z16 + (colA + 1), it], acc[3])
                    plsc.store_scatter(stage_s, [z16 + (colA + 8), it], acc[4])
                    plsc.store_scatter(stage_s, [z16 + (colA + 9), it], acc[5])

                return c2

            lax.fori_loop(0, _L, lvl_acc, 0)
            pltpu.sync_copy(stage_s, out_h.at[:, pl.ds(base + po, _CPTS)])

        compute_indices(0, 0)
        fire(0)

        def step(kc, buf):
            nxt = jnp.minimum(kc + 1, _NCH - 1)
            compute_indices(nxt, 1 - buf)
            fire(1 - buf)
            for i in range(_L):
                pltpu.make_async_copy(
                    tbl_h.at[idx_s.at[buf, i]], rows_s.at[buf, i], sems[buf]
                ).wait()
            accumulate(kc, buf)

        def body(kc2, c0):
            step(kc2 * 2, 0)
            step(kc2 * 2 + 1, 1)
            return c0

        lax.fori_loop(0, _NCH // 2, body, 0)
        for i in range(_L):
            pltpu.make_async_copy(
                tbl_h.at[idx_s.at[0, i]], rows_s.at[0, i], sems[0]
            ).wait()

    return k(xs, ys, zs, tbl, resv, parv)


_BP = 2048


def _mlp_body(sc_ref, dirt_ref, w1t_ref, w2t_ref, c1t_ref, c2t_ref, c3t_ref,
              sig_ref, col_ref):
    featt = sc_ref[:96, :]
    h1t = jnp.maximum(jnp.dot(w1t_ref[...], featt,
                              preferred_element_type=jnp.float32), 0.0)
    ht = jnp.dot(w2t_ref[...], h1t, preferred_element_type=jnp.float32)
    sig_ref[...] = jnp.exp(ht[0:1, :])

    d = dirt_ref[...]
    x = d[0:1, :]
    y = d[1:2, :]
    z = d[2:3, :]
    inv = 1.0 / (jnp.sqrt(x * x + y * y + z * z) + 1e-8)
    x = x * inv
    y = y * inv
    z = z * inv
    x2, y2, z2 = x * x, y * y, z * z
    xy, yz, xz = x * y, y * z, x * z
    comps = [
        0.28209479177387814 * jnp.ones_like(x),
        -0.48860251190291987 * y,
        0.48860251190291987 * z,
        -0.48860251190291987 * x,
        1.0925484305920792 * xy,
        -1.0925484305920792 * yz,
        0.94617469575755997 * z2 - 0.31539156525252005,
        -1.0925484305920792 * xz,
        0.54627421529603959 * (x2 - y2),
        -0.59004358992664352 * y * (3.0 * x2 - y2),
        2.8906114426405538 * xy * z,
        -0.45704579946446572 * y * (4.0 * z2 - x2 - y2),
        0.3731763325901154 * z * (2.0 * z2 - 3.0 * x2 - 3.0 * y2),
        -0.45704579946446572 * x * (4.0 * z2 - x2 - y2),
        1.4453057213202769 * z * (x2 - y2),
        -0.59004358992664352 * x * (x2 - 3.0 * y2),
    ]
    sht = jnp.concatenate(comps, axis=0)
    ci1 = (jnp.dot(c1t_ref[:, :16], sht, preferred_element_type=jnp.float32)
           + jnp.dot(c1t_ref[:, 16:], ht, preferred_element_type=jnp.float32))
    cc = jnp.maximum(ci1, 0.0)
    cc = jnp.maximum(jnp.dot(c2t_ref[...], cc, preferred_element_type=jnp.float32), 0.0)
    col_ref[...] = jax.nn.sigmoid(
        jnp.dot(c3t_ref[...], cc, preferred_element_type=jnp.float32))


def _tc_mlp(sc_out, dirt, W1t, W2t, C1t, C2t, C3t):
    grid = (_NPTS // _BP,)
    return pl.pallas_call(
        _mlp_body,
        grid=grid,
        in_specs=[
            pl.BlockSpec((_OC, _BP), lambda i: (0, i)),
            pl.BlockSpec((3, _BP), lambda i: (0, i)),
            pl.BlockSpec((64, 96), lambda i: (0, 0)),
            pl.BlockSpec((16, 64), lambda i: (0, 0)),
            pl.BlockSpec((64, 32), lambda i: (0, 0)),
            pl.BlockSpec((64, 64), lambda i: (0, 0)),
            pl.BlockSpec((3, 64), lambda i: (0, 0)),
        ],
        out_specs=[
            pl.BlockSpec((1, _BP), lambda i: (0, i)),
            pl.BlockSpec((3, _BP), lambda i: (0, i)),
        ],
        out_shape=[
            jax.ShapeDtypeStruct((1, _NPTS), jnp.float32),
            jax.ShapeDtypeStruct((3, _NPTS), jnp.float32),
        ],
    )(sc_out, dirt, W1t, W2t, C1t, C2t, C3t)


def kernel(original_xyzs, dirs, static_table, tableA, tableB, table2A, table2B,
           W1, W2, C1, C2, C3):
    xs = original_xyzs[:, 0]
    ys = original_xyzs[:, 1]
    zs = original_xyzs[:, 2]
    t0 = original_xyzs[0, 3]

    prev1 = 1.0 - (t0 * 16.0 - 8.0)
    nxt1 = 1.0 - prev1
    s1 = prev1 + nxt1
    prev2 = 1.0 - (t0 * 20.0 - 10.0)
    nxt2 = 1.0 - prev2
    s2 = prev2 + nxt2
    par = jnp.concatenate([
        jnp.stack([prev1 / s1, nxt1 / s1, prev2 / s2, nxt2 / s2]),
        jnp.zeros((12,), jnp.float32),
    ])
    resv = jnp.asarray(_RES, dtype=jnp.float32)

    def _phys(t):
        return t.reshape(_L, _T // 128, 128, _F).transpose(0, 1, 3, 2).reshape(-1)

    tbl = _sc_pack(_phys(static_table), _phys(tableA), _phys(tableB),
                   _phys(table2A), _phys(table2B))

    sc_out = _sc_encode(xs, ys, zs, tbl, resv, par)

    sigt, colt = _tc_mlp(sc_out, dirs.T, W1.T, W2.T, C1.T, C2.T, C3.T)
    sigma = sigt.reshape(_NPTS)
    color = colt.T
    af1 = sc_out[96:104, :].T
    af2 = sc_out[104:112, :].T
    return (sigma, color, af1, af2)


# speedup vs baseline: 46.6839x; 1.0840x over previous
import functools

import numpy as np
import jax
import jax.numpy as jnp
from jax import lax
from jax.experimental import pallas as pl
from jax.experimental.pallas import tpu as pltpu
from jax.experimental.pallas import tpu_sc as plsc

_L = 16
_F = 2
_T = 2 ** 19
_NPTS = 65536
_B = float(np.exp(np.log(4096.0 / 16.0) / (_L - 1)))
_RES = [int(np.floor(16 * (_B ** l))) for l in range(_L)]
_P2I = int(np.uint32(2654435761).view(np.int32))
_P3I = int(np.uint32(805459861).view(np.int32))

_NW = 32
_PPW = _NPTS // _NW
_CPTS = 16
_NCH = _PPW // _CPTS
_D = 16
_OC = 112


_PR = 2048
_PNCH = (_T // 2) // _PR


def _sc_pack(t0, t1, t2, t3, t4):
    mesh = plsc.VectorSubcoreMesh(core_axis_name="c", subcore_axis_name="s")

    @functools.partial(
        pl.kernel,
        mesh=mesh,
        out_type=jax.ShapeDtypeStruct((_L * _T, _D), jnp.float32),
        compiler_params=pltpu.CompilerParams(
            needs_layout_passes=False, use_tc_tiling_on_sc=False),
        scratch_types=[
            pltpu.VMEM((2, 5, _PR * _F), jnp.float32),
            pltpu.VMEM((2, _PR, _D), jnp.float32),
            pltpu.SemaphoreType.DMA,
            pltpu.SemaphoreType.DMA,
            pltpu.SemaphoreType.DMA,
            pltpu.SemaphoreType.DMA,
        ],
    )
    def k(t0_h, t1_h, t2_h, t3_h, t4_h, out_h, in_s, out_s,
          semi0, semi1, semo0, semo1):
        tabs = (t0_h, t1_h, t2_h, t3_h, t4_h)
        semi = (semi0, semi1)
        semo = (semo0, semo1)
        wid = lax.axis_index("s") * 2 + lax.axis_index("c")
        lvl = wid // 2
        h0 = (wid % 2) * (_T // 2)
        it = lax.iota(jnp.int32, 16)
        z16 = it * 0

        def fire_in(kc, buf):
            e0 = lvl * (_T * _F) + ((h0 + kc * _PR) // 128) * 256
            return [
                pltpu.async_copy(tabs[t].at[pl.ds(e0, _PR * _F)],
                                 in_s.at[buf, t], semi[buf])
                for t in range(5)
            ]

        def drain_in(kc, buf):
            e0 = lvl * (_T * _F) + ((h0 + kc * _PR) // 128) * 256
            for t in range(5):
                pltpu.make_async_copy(tabs[t].at[pl.ds(e0, _PR * _F)],
                                      in_s.at[buf, t], semi[buf]).wait()

        def out_slice(kc):
            return out_h.at[pl.ds((lvl * _T + h0 + kc * _PR), _PR), :]

        def interleave(kc, buf):
            bvec = z16 + buf

            def gbody(bc, c0):
                blk = bc // 2
                c = bc - blk * 2
                for rr in range(8):
                    rv = blk * 128 + rr * 16 + it
                    src = blk * 256 + c * 128 + rr * 16
                    for t in range(5):
                        chv = z16 + (2 * t + c)
                        v = in_s[buf, t, pl.ds(src, 16)]
                        plsc.store_scatter(out_s, [bvec, rv, chv], v)
                return c0

            lax.fori_loop(0, (_PR // 128) * 2, gbody, 0)

        def step(kc, buf, first):
            nxt = jnp.minimum(kc + 1, _PNCH - 1)
            fire_in(nxt, 1 - buf)
            drain_in(kc, buf)
            if not first:
                pltpu.make_async_copy(out_s.at[buf],
                                      out_slice(jnp.maximum(kc - 2, 0)),
                                      semo[buf]).wait()
            interleave(kc, buf)
            pltpu.async_copy(out_s.at[buf], out_slice(kc), semo[buf])

        fire_in(0, 0)
        step(0, 0, True)
        step(1, 1, True)

        def body(kc2, c0):
            step(kc2 * 2, 0, False)
            step(kc2 * 2 + 1, 1, False)
            return c0

        lax.fori_loop(1, _PNCH // 2, body, 0)
        drain_in(_PNCH - 1, 0)
        pltpu.make_async_copy(out_s.at[0], out_slice(_PNCH - 2), semo[0]).wait()
        pltpu.make_async_copy(out_s.at[1], out_slice(_PNCH - 1), semo[1]).wait()

    return k(t0, t1, t2, t3, t4)


def _sc_encode(xs, ys, zs, tbl, resv, parv):
    mesh = plsc.VectorSubcoreMesh(core_axis_name="c", subcore_axis_name="s")

    @functools.partial(
        pl.kernel,
        mesh=mesh,
        out_type=jax.ShapeDtypeStruct((_OC, _NPTS), jnp.float32),
        compiler_params=pltpu.CompilerParams(
            needs_layout_passes=False, use_tc_tiling_on_sc=False),
        scratch_types=[
            pltpu.VMEM((_PPW,), jnp.float32),
            pltpu.VMEM((_PPW,), jnp.float32),
            pltpu.VMEM((_PPW,), jnp.float32),
            pltpu.VMEM((16,), jnp.float32),
            pltpu.VMEM((16,), jnp.float32),
            pltpu.VMEM((2, _L, 8 * _CPTS), jnp.int32),
            pltpu.VMEM((2, _L, 8 * _CPTS), jnp.float32),
            pltpu.VMEM((2, _L, 8 * _CPTS, _D), jnp.float32),
            pltpu.VMEM((_OC, _CPTS), jnp.float32),
            pltpu.SemaphoreType.DMA,
            pltpu.SemaphoreType.DMA,
        ],
    )
    def k(xs_h, ys_h, zs_h, tbl_h, res_h, par_h, out_h,
          x_s, y_s, z_s, res_s, par_s, idx_s, w_s, rows_s, stage_s, sem0, sem1):
        wid = lax.axis_index("s") * 2 + lax.axis_index("c")
        base = wid * _PPW
        pltpu.sync_copy(xs_h.at[pl.ds(base, _PPW)], x_s)
        pltpu.sync_copy(ys_h.at[pl.ds(base, _PPW)], y_s)
        pltpu.sync_copy(zs_h.at[pl.ds(base, _PPW)], z_s)
        pltpu.sync_copy(res_h, res_s)
        pltpu.sync_copy(par_h, par_s)
        it = lax.iota(jnp.int32, 16)
        z16 = it * 0
        a1 = plsc.load_gather(par_s, [z16])
        b1 = plsc.load_gather(par_s, [z16 + 1])
        a2 = plsc.load_gather(par_s, [z16 + 2])
        b2 = plsc.load_gather(par_s, [z16 + 3])
        sems = (sem0, sem1)
        cvecs = [z16 + j for j in range(10)]
        rvecs = [c * _CPTS + it for c in range(8)]

        def compute_indices(kc, buf):
            po = kc * _CPTS
            x = x_s[pl.ds(po, _CPTS)] * 0.5 + 0.5
            y = y_s[pl.ds(po, _CPTS)] * 0.5 + 0.5
            z = z_s[pl.ds(po, _CPTS)] * 0.5 + 0.5

            def lvl_idx(l, c2):
                r = plsc.load_gather(res_s, [z16 + l])
                px = x * r
                py = y * r
                pz = z * r
                ix = px.astype(jnp.int32)
                iy = py.astype(jnp.int32)
                iz = pz.astype(jnp.int32)
                fx = px - ix.astype(jnp.float32)
                fy = py - iy.astype(jnp.float32)
                fz = pz - iz.astype(jnp.float32)
                gy = iy * _P2I
                gz = iz * _P3I
                lT = l * _T
                for c in range(8):
                    cx, cy, cz = c & 1, (c >> 1) & 1, (c >> 2) & 1
                    hx = ix + cx if cx else ix
                    hy = gy + _P2I if cy else gy
                    hz = gz + _P3I if cz else gz
                    h = ((hx ^ hy) ^ hz) & (_T - 1)
                    wx = fx if cx else 1.0 - fx
                    wy = fy if cy else 1.0 - fy
                    wz = fz if cz else 1.0 - fz
                    idx_s[buf, l, pl.ds(c * _CPTS, _CPTS)] = h + lT
                    w_s[buf, l, pl.ds(c * _CPTS, _CPTS)] = wx * wy * wz
                return c2

            lax.fori_loop(0, _L, lvl_idx, 0)

        def fire(buf):
            return [
                pltpu.async_copy(tbl_h.at[idx_s.at[buf, i]], rows_s.at[buf, i],
                                 sems[buf])
                for i in range(_L)
            ]

        def accumulate(kc, buf):
            po = kc * _CPTS

            def lvl_acc(l, c2):
                bvec = z16 + buf
                lvec = z16 + l
                acc = [jnp.zeros((16,), jnp.float32) for _ in range(10)]
                for c in range(8):
                    w = w_s[buf, l, pl.ds(c * _CPTS, _CPTS)]
                    for j in range(10):
                        v = plsc.load_gather(rows_s, [bvec, lvec, rvecs[c], cvecs[j]])
                        acc[j] = acc[j] + w * v
                col = 2 * l
                plsc.store_scatter(stage_s, [z16 + col, it], acc[0])
                plsc.store_scatter(stage_s, [z16 + (col + 1), it], acc[1])
                plsc.store_scatter(stage_s, [z16 + (32 + col), it], a1 * acc[2] + b1 * acc[4])
                plsc.store_scatter(stage_s, [z16 + (33 + col), it], a1 * acc[3] + b1 * acc[5])
                plsc.store_scatter(stage_s, [z16 + (64 + col), it], a2 * acc[6] + b2 * acc[8])
                plsc.store_scatter(stage_s, [z16 + (65 + col), it], a2 * acc[7] + b2 * acc[9])

                @pl.when(l >= 12)
                def _():
                    colA = 96 + 2 * (l - 12)
                    plsc.store_scatter(stage_s, [z16 + colA, it], acc[2])
                    plsc.store_scatter(stage_s, [z16 + (colA + 1), it], acc[3])
                    plsc.store_scatter(stage_s, [z16 + (colA + 8), it], acc[4])
                    plsc.store_scatter(stage_s, [z16 + (colA + 9), it], acc[5])

                return c2

            lax.fori_loop(0, _L, lvl_acc, 0)
            pltpu.sync_copy(stage_s, out_h.at[:, pl.ds(base + po, _CPTS)])

        compute_indices(0, 0)
        fire(0)

        def step(kc, buf):
            nxt = jnp.minimum(kc + 1, _NCH - 1)
            compute_indices(nxt, 1 - buf)
            fire(1 - buf)
            for i in range(_L):
                pltpu.make_async_copy(
                    tbl_h.at[idx_s.at[buf, i]], rows_s.at[buf, i], sems[buf]
                ).wait()
            accumulate(kc, buf)

        def body(kc2, c0):
            step(kc2 * 2, 0)
            step(kc2 * 2 + 1, 1)
            return c0

        lax.fori_loop(0, _NCH // 2, body, 0)
        for i in range(_L):
            pltpu.make_async_copy(
                tbl_h.at[idx_s.at[0, i]], rows_s.at[0, i], sems[0]
            ).wait()

    return k(xs, ys, zs, tbl, resv, parv)


_BP = 2048


def _mlp_body(sc_ref, dirt_ref, w1t_ref, w2t_ref, c1t_ref, c2t_ref, c3t_ref,
              sig_ref, col_ref):
    featt = sc_ref[:96, :]
    h1t = jnp.maximum(jnp.dot(w1t_ref[...], featt,
                              preferred_element_type=jnp.float32), 0.0)
    ht = jnp.dot(w2t_ref[...], h1t, preferred_element_type=jnp.float32)
    sig_ref[...] = jnp.exp(ht[0:1, :])

    d = dirt_ref[...]
    x = d[0:1, :]
    y = d[1:2, :]
    z = d[2:3, :]
    inv = 1.0 / (jnp.sqrt(x * x + y * y + z * z) + 1e-8)
    x = x * inv
    y = y * inv
    z = z * inv
    x2, y2, z2 = x * x, y * y, z * z
    xy, yz, xz = x * y, y * z, x * z
    comps = [
        0.28209479177387814 * jnp.ones_like(x),
        -0.48860251190291987 * y,
        0.48860251190291987 * z,
        -0.48860251190291987 * x,
        1.0925484305920792 * xy,
        -1.0925484305920792 * yz,
        0.94617469575755997 * z2 - 0.31539156525252005,
        -1.0925484305920792 * xz,
        0.54627421529603959 * (x2 - y2),
        -0.59004358992664352 * y * (3.0 * x2 - y2),
        2.8906114426405538 * xy * z,
        -0.45704579946446572 * y * (4.0 * z2 - x2 - y2),
        0.3731763325901154 * z * (2.0 * z2 - 3.0 * x2 - 3.0 * y2),
        -0.45704579946446572 * x * (4.0 * z2 - x2 - y2),
        1.4453057213202769 * z * (x2 - y2),
        -0.59004358992664352 * x * (x2 - 3.0 * y2),
    ]
    sht = jnp.concatenate(comps, axis=0)
    ci1 = (jnp.dot(c1t_ref[:, :16], sht, preferred_element_type=jnp.float32)
           + jnp.dot(c1t_ref[:, 16:], ht, preferred_element_type=jnp.float32))
    cc = jnp.maximum(ci1, 0.0)
    cc = jnp.maximum(jnp.dot(c2t_ref[...], cc, preferred_element_type=jnp.float32), 0.0)
    col_ref[...] = jax.nn.sigmoid(
        jnp.dot(c3t_ref[...], cc, preferred_element_type=jnp.float32))


def _tc_mlp(sc_out, dirt, W1t, W2t, C1t, C2t, C3t):
    grid = (_NPTS // _BP,)
    return pl.pallas_call(
        _mlp_body,
        grid=grid,
        in_specs=[
            pl.BlockSpec((_OC, _BP), lambda i: (0, i)),
            pl.BlockSpec((3, _BP), lambda i: (0, i)),
            pl.BlockSpec((64, 96), lambda i: (0, 0)),
            pl.BlockSpec((16, 64), lambda i: (0, 0)),
            pl.BlockSpec((64, 32), lambda i: (0, 0)),
            pl.BlockSpec((64, 64), lambda i: (0, 0)),
            pl.BlockSpec((3, 64), lambda i: (0, 0)),
        ],
        out_specs=[
            pl.BlockSpec((1, _BP), lambda i: (0, i)),
            pl.BlockSpec((3, _BP), lambda i: (0, i)),
        ],
        out_shape=[
            jax.ShapeDtypeStruct((1, _NPTS), jnp.float32),
            jax.ShapeDtypeStruct((3, _NPTS), jnp.float32),
        ],
    )(sc_out, dirt, W1t, W2t, C1t, C2t, C3t)


def kernel(original_xyzs, dirs, static_table, tableA, tableB, table2A, table2B,
           W1, W2, C1, C2, C3):
    xs = original_xyzs[:, 0]
    ys = original_xyzs[:, 1]
    zs = original_xyzs[:, 2]
    t0 = original_xyzs[0, 3]

    prev1 = 1.0 - (t0 * 16.0 - 8.0)
    nxt1 = 1.0 - prev1
    s1 = prev1 + nxt1
    prev2 = 1.0 - (t0 * 20.0 - 10.0)
    nxt2 = 1.0 - prev2
    s2 = prev2 + nxt2
    par = jnp.concatenate([
        jnp.stack([prev1 / s1, nxt1 / s1, prev2 / s2, nxt2 / s2]),
        jnp.zeros((12,), jnp.float32),
    ])
    resv = jnp.asarray(_RES, dtype=jnp.float32)

    def _phys(t):
        return t.reshape(_L, _T // 128, 128, _F).transpose(0, 1, 3, 2).reshape(-1)

    tbl = _sc_pack(_phys(static_table), _phys(tableA), _phys(tableB),
                   _phys(table2A), _phys(table2B))

    sc_out = _sc_encode(xs, ys, zs, tbl, resv, par)

    sigt, colt = _tc_mlp(sc_out, dirs.T, W1.T, W2.T, C1.T, C2.T, C3.T)
    sigma = sigt.reshape(_NPTS)
    color = colt.T
    af1 = sc_out[96:104, :].T
    af2 = sc_out[104:112, :].T
    return (sigma, color, af1, af2)


# speedup vs baseline: 72.6597x; 1.5564x over previous
import functools

import numpy as np
import jax
import jax.numpy as jnp
from jax import lax
from jax.experimental import pallas as pl
from jax.experimental.pallas import tpu as pltpu
from jax.experimental.pallas import tpu_sc as plsc

_L = 16
_F = 2
_T = 2 ** 19
_NPTS = 65536
_B = float(np.exp(np.log(4096.0 / 16.0) / (_L - 1)))
_RES = [int(np.floor(16 * (_B ** l))) for l in range(_L)]
_P2I = int(np.uint32(2654435761).view(np.int32))
_P3I = int(np.uint32(805459861).view(np.int32))

_NW = 32
_PPW = _NPTS // _NW
_CPTS = 16
_NCH = _PPW // _CPTS
_D = 16
_OC = 112


_PR = 2048
_PNCH = (_T // 2) // _PR


def _sc_pack(t0, t1, t2, t3, t4, parv):
    mesh = plsc.VectorSubcoreMesh(core_axis_name="c", subcore_axis_name="s")

    @functools.partial(
        pl.kernel,
        mesh=mesh,
        out_type=jax.ShapeDtypeStruct((_L * _T, _D), jnp.float32),
        compiler_params=pltpu.CompilerParams(
            needs_layout_passes=False, use_tc_tiling_on_sc=False),
        scratch_types=[
            pltpu.VMEM((2, 5, _PR * _F), jnp.float32),
            pltpu.VMEM((2, _PR, _D), jnp.float32),
            pltpu.VMEM((16,), jnp.float32),
            pltpu.SemaphoreType.DMA,
            pltpu.SemaphoreType.DMA,
            pltpu.SemaphoreType.DMA,
            pltpu.SemaphoreType.DMA,
        ],
    )
    def k(t0_h, t1_h, t2_h, t3_h, t4_h, par_h, out_h, in_s, out_s, par_s,
          semi0, semi1, semo0, semo1):
        tabs = (t0_h, t1_h, t2_h, t3_h, t4_h)
        semi = (semi0, semi1)
        semo = (semo0, semo1)
        wid = lax.axis_index("s") * 2 + lax.axis_index("c")
        lvl = wid // 2
        h0 = (wid % 2) * (_T // 2)
        it = lax.iota(jnp.int32, 16)
        z16 = it * 0
        pltpu.sync_copy(par_h, par_s)
        a1 = plsc.load_gather(par_s, [z16])
        b1 = plsc.load_gather(par_s, [z16 + 1])
        a2 = plsc.load_gather(par_s, [z16 + 2])
        b2 = plsc.load_gather(par_s, [z16 + 3])

        def fire_in(kc, buf):
            e0 = lvl * (_T * _F) + ((h0 + kc * _PR) // 128) * 256
            return [
                pltpu.async_copy(tabs[t].at[pl.ds(e0, _PR * _F)],
                                 in_s.at[buf, t], semi[buf])
                for t in range(5)
            ]

        def drain_in(kc, buf):
            e0 = lvl * (_T * _F) + ((h0 + kc * _PR) // 128) * 256
            for t in range(5):
                pltpu.make_async_copy(tabs[t].at[pl.ds(e0, _PR * _F)],
                                      in_s.at[buf, t], semi[buf]).wait()

        def out_slice(kc):
            return out_h.at[pl.ds((lvl * _T + h0 + kc * _PR), _PR), :]

        def interleave(kc, buf):
            bvec = z16 + buf

            def gbody(bc, c0):
                blk = bc // 2
                c = bc - blk * 2
                for rr in range(8):
                    rv = blk * 128 + rr * 16 + it
                    src = blk * 256 + c * 128 + rr * 16
                    vs = [in_s[buf, t, pl.ds(src, 16)] for t in range(5)]
                    plsc.store_scatter(out_s, [bvec, rv, z16 + c], vs[0])
                    plsc.store_scatter(out_s, [bvec, rv, z16 + (2 + c)],
                                       a1 * vs[1] + b1 * vs[2])
                    plsc.store_scatter(out_s, [bvec, rv, z16 + (4 + c)],
                                       a2 * vs[3] + b2 * vs[4])
                    plsc.store_scatter(out_s, [bvec, rv, z16 + (6 + c)], vs[1])
                    plsc.store_scatter(out_s, [bvec, rv, z16 + (8 + c)], vs[2])
                return c0

            lax.fori_loop(0, (_PR // 128) * 2, gbody, 0)

        def step(kc, buf, first):
            nxt = jnp.minimum(kc + 1, _PNCH - 1)
            fire_in(nxt, 1 - buf)
            drain_in(kc, buf)
            if not first:
                pltpu.make_async_copy(out_s.at[buf],
                                      out_slice(jnp.maximum(kc - 2, 0)),
                                      semo[buf]).wait()
            interleave(kc, buf)
            pltpu.async_copy(out_s.at[buf], out_slice(kc), semo[buf])

        fire_in(0, 0)
        step(0, 0, True)
        step(1, 1, True)

        def body(kc2, c0):
            step(kc2 * 2, 0, False)
            step(kc2 * 2 + 1, 1, False)
            return c0

        lax.fori_loop(1, _PNCH // 2, body, 0)
        drain_in(_PNCH - 1, 0)
        pltpu.make_async_copy(out_s.at[0], out_slice(_PNCH - 2), semo[0]).wait()
        pltpu.make_async_copy(out_s.at[1], out_slice(_PNCH - 1), semo[1]).wait()

    return k(t0, t1, t2, t3, t4, parv)


def _sc_encode(xs, ys, zs, tbl, resv, parv):
    mesh = plsc.VectorSubcoreMesh(core_axis_name="c", subcore_axis_name="s")

    @functools.partial(
        pl.kernel,
        mesh=mesh,
        out_type=jax.ShapeDtypeStruct((_OC, _NPTS), jnp.float32),
        compiler_params=pltpu.CompilerParams(
            needs_layout_passes=False, use_tc_tiling_on_sc=False),
        scratch_types=[
            pltpu.VMEM((_PPW,), jnp.float32),
            pltpu.VMEM((_PPW,), jnp.float32),
            pltpu.VMEM((_PPW,), jnp.float32),
            pltpu.VMEM((16,), jnp.float32),
            pltpu.VMEM((16,), jnp.float32),
            pltpu.VMEM((2, _L, 8 * _CPTS), jnp.int32),
            pltpu.VMEM((2, _L, 8 * _CPTS), jnp.float32),
            pltpu.VMEM((2, _L, 8 * _CPTS, _D), jnp.float32),
            pltpu.VMEM((_OC, _CPTS), jnp.float32),
            pltpu.SemaphoreType.DMA,
            pltpu.SemaphoreType.DMA,
        ],
    )
    def k(xs_h, ys_h, zs_h, tbl_h, res_h, par_h, out_h,
          x_s, y_s, z_s, res_s, par_s, idx_s, w_s, rows_s, stage_s, sem0, sem1):
        wid = lax.axis_index("s") * 2 + lax.axis_index("c")
        base = wid * _PPW
        pltpu.sync_copy(xs_h.at[pl.ds(base, _PPW)], x_s)
        pltpu.sync_copy(ys_h.at[pl.ds(base, _PPW)], y_s)
        pltpu.sync_copy(zs_h.at[pl.ds(base, _PPW)], z_s)
        pltpu.sync_copy(res_h, res_s)
        pltpu.sync_copy(par_h, par_s)
        it = lax.iota(jnp.int32, 16)
        z16 = it * 0
        a1 = plsc.load_gather(par_s, [z16])
        b1 = plsc.load_gather(par_s, [z16 + 1])
        a2 = plsc.load_gather(par_s, [z16 + 2])
        b2 = plsc.load_gather(par_s, [z16 + 3])
        sems = (sem0, sem1)
        cvecs = [z16 + j for j in range(10)]
        rvecs = [c * _CPTS + it for c in range(8)]

        def compute_indices(kc, buf):
            po = kc * _CPTS
            x = x_s[pl.ds(po, _CPTS)] * 0.5 + 0.5
            y = y_s[pl.ds(po, _CPTS)] * 0.5 + 0.5
            z = z_s[pl.ds(po, _CPTS)] * 0.5 + 0.5

            def lvl_idx(l, c2):
                r = plsc.load_gather(res_s, [z16 + l])
                px = x * r
                py = y * r
                pz = z * r
                ix = px.astype(jnp.int32)
                iy = py.astype(jnp.int32)
                iz = pz.astype(jnp.int32)
                fx = px - ix.astype(jnp.float32)
                fy = py - iy.astype(jnp.float32)
                fz = pz - iz.astype(jnp.float32)
                gy = iy * _P2I
                gz = iz * _P3I
                lT = l * _T
                for c in range(8):
                    cx, cy, cz = c & 1, (c >> 1) & 1, (c >> 2) & 1
                    hx = ix + cx if cx else ix
                    hy = gy + _P2I if cy else gy
                    hz = gz + _P3I if cz else gz
                    h = ((hx ^ hy) ^ hz) & (_T - 1)
                    wx = fx if cx else 1.0 - fx
                    wy = fy if cy else 1.0 - fy
                    wz = fz if cz else 1.0 - fz
                    idx_s[buf, l, pl.ds(c * _CPTS, _CPTS)] = h + lT
                    w_s[buf, l, pl.ds(c * _CPTS, _CPTS)] = wx * wy * wz
                return c2

            lax.fori_loop(0, _L, lvl_idx, 0)

        def fire(buf):
            return [
                pltpu.async_copy(tbl_h.at[idx_s.at[buf, i]], rows_s.at[buf, i],
                                 sems[buf])
                for i in range(_L)
            ]

        def accumulate(kc, buf):
            po = kc * _CPTS

            def lvl_acc(l, nj, c2):
                bvec = z16 + buf
                lvec = z16 + l
                acc = [jnp.zeros((16,), jnp.float32) for _ in range(nj)]
                for c in range(8):
                    w = w_s[buf, l, pl.ds(c * _CPTS, _CPTS)]
                    for j in range(nj):
                        v = plsc.load_gather(rows_s, [bvec, lvec, rvecs[c], cvecs[j]])
                        acc[j] = acc[j] + w * v
                col = 2 * l
                plsc.store_scatter(stage_s, [z16 + col, it], acc[0])
                plsc.store_scatter(stage_s, [z16 + (col + 1), it], acc[1])
                plsc.store_scatter(stage_s, [z16 + (32 + col), it], acc[2])
                plsc.store_scatter(stage_s, [z16 + (33 + col), it], acc[3])
                plsc.store_scatter(stage_s, [z16 + (64 + col), it], acc[4])
                plsc.store_scatter(stage_s, [z16 + (65 + col), it], acc[5])
                if nj == 10:
                    colA = 96 + 2 * (l - 12)
                    plsc.store_scatter(stage_s, [z16 + colA, it], acc[6])
                    plsc.store_scatter(stage_s, [z16 + (colA + 1), it], acc[7])
                    plsc.store_scatter(stage_s, [z16 + (colA + 8), it], acc[8])
                    plsc.store_scatter(stage_s, [z16 + (colA + 9), it], acc[9])
                return c2

            lax.fori_loop(0, 12, lambda l, c2: lvl_acc(l, 6, c2), 0)
            lax.fori_loop(12, _L, lambda l, c2: lvl_acc(l, 10, c2), 0)
            pltpu.sync_copy(stage_s, out_h.at[:, pl.ds(base + po, _CPTS)])

        compute_indices(0, 0)
        fire(0)

        def step(kc, buf):
            nxt = jnp.minimum(kc + 1, _NCH - 1)
            compute_indices(nxt, 1 - buf)
            fire(1 - buf)
            for i in range(_L):
                pltpu.make_async_copy(
                    tbl_h.at[idx_s.at[buf, i]], rows_s.at[buf, i], sems[buf]
                ).wait()
            accumulate(kc, buf)

        def body(kc2, c0):
            step(kc2 * 2, 0)
            step(kc2 * 2 + 1, 1)
            return c0

        lax.fori_loop(0, _NCH // 2, body, 0)
        for i in range(_L):
            pltpu.make_async_copy(
                tbl_h.at[idx_s.at[0, i]], rows_s.at[0, i], sems[0]
            ).wait()

    return k(xs, ys, zs, tbl, resv, parv)


_BP = 2048


def _mlp_body(sc_ref, dirt_ref, w1t_ref, w2t_ref, c1t_ref, c2t_ref, c3t_ref,
              sig_ref, col_ref):
    featt = sc_ref[:96, :]
    h1t = jnp.maximum(jnp.dot(w1t_ref[...], featt,
                              preferred_element_type=jnp.float32), 0.0)
    ht = jnp.dot(w2t_ref[...], h1t, preferred_element_type=jnp.float32)
    sig_ref[...] = jnp.exp(ht[0:1, :])

    d = dirt_ref[...]
    x = d[0:1, :]
    y = d[1:2, :]
    z = d[2:3, :]
    inv = 1.0 / (jnp.sqrt(x * x + y * y + z * z) + 1e-8)
    x = x * inv
    y = y * inv
    z = z * inv
    x2, y2, z2 = x * x, y * y, z * z
    xy, yz, xz = x * y, y * z, x * z
    comps = [
        0.28209479177387814 * jnp.ones_like(x),
        -0.48860251190291987 * y,
        0.48860251190291987 * z,
        -0.48860251190291987 * x,
        1.0925484305920792 * xy,
        -1.0925484305920792 * yz,
        0.94617469575755997 * z2 - 0.31539156525252005,
        -1.0925484305920792 * xz,
        0.54627421529603959 * (x2 - y2),
        -0.59004358992664352 * y * (3.0 * x2 - y2),
        2.8906114426405538 * xy * z,
        -0.45704579946446572 * y * (4.0 * z2 - x2 - y2),
        0.3731763325901154 * z * (2.0 * z2 - 3.0 * x2 - 3.0 * y2),
        -0.45704579946446572 * x * (4.0 * z2 - x2 - y2),
        1.4453057213202769 * z * (x2 - y2),
        -0.59004358992664352 * x * (x2 - 3.0 * y2),
    ]
    sht = jnp.concatenate(comps, axis=0)
    ci1 = (jnp.dot(c1t_ref[:, :16], sht, preferred_element_type=jnp.float32)
           + jnp.dot(c1t_ref[:, 16:], ht, preferred_element_type=jnp.float32))
    cc = jnp.maximum(ci1, 0.0)
    cc = jnp.maximum(jnp.dot(c2t_ref[...], cc, preferred_element_type=jnp.float32), 0.0)
    col_ref[...] = jax.nn.sigmoid(
        jnp.dot(c3t_ref[...], cc, preferred_element_type=jnp.float32))


def _tc_mlp(sc_out, dirt, W1t, W2t, C1t, C2t, C3t):
    grid = (_NPTS // _BP,)
    return pl.pallas_call(
        _mlp_body,
        grid=grid,
        in_specs=[
            pl.BlockSpec((_OC, _BP), lambda i: (0, i)),
            pl.BlockSpec((3, _BP), lambda i: (0, i)),
            pl.BlockSpec((64, 96), lambda i: (0, 0)),
            pl.BlockSpec((16, 64), lambda i: (0, 0)),
            pl.BlockSpec((64, 32), lambda i: (0, 0)),
            pl.BlockSpec((64, 64), lambda i: (0, 0)),
            pl.BlockSpec((3, 64), lambda i: (0, 0)),
        ],
        out_specs=[
            pl.BlockSpec((1, _BP), lambda i: (0, i)),
            pl.BlockSpec((3, _BP), lambda i: (0, i)),
        ],
        out_shape=[
            jax.ShapeDtypeStruct((1, _NPTS), jnp.float32),
            jax.ShapeDtypeStruct((3, _NPTS), jnp.float32),
        ],
    )(sc_out, dirt, W1t, W2t, C1t, C2t, C3t)


def kernel(original_xyzs, dirs, static_table, tableA, tableB, table2A, table2B,
           W1, W2, C1, C2, C3):
    xs = original_xyzs[:, 0]
    ys = original_xyzs[:, 1]
    zs = original_xyzs[:, 2]
    t0 = original_xyzs[0, 3]

    prev1 = 1.0 - (t0 * 16.0 - 8.0)
    nxt1 = 1.0 - prev1
    s1 = prev1 + nxt1
    prev2 = 1.0 - (t0 * 20.0 - 10.0)
    nxt2 = 1.0 - prev2
    s2 = prev2 + nxt2
    par = jnp.concatenate([
        jnp.stack([prev1 / s1, nxt1 / s1, prev2 / s2, nxt2 / s2]),
        jnp.zeros((12,), jnp.float32),
    ])
    resv = jnp.asarray(_RES, dtype=jnp.float32)

    def _phys(t):
        return t.reshape(_L, _T // 128, 128, _F).transpose(0, 1, 3, 2).reshape(-1)

    tbl = _sc_pack(_phys(static_table), _phys(tableA), _phys(tableB),
                   _phys(table2A), _phys(table2B), par)

    sc_out = _sc_encode(xs, ys, zs, tbl, resv, par)

    sigt, colt = _tc_mlp(sc_out, dirs.T, W1.T, W2.T, C1.T, C2.T, C3.T)
    sigma = sigt.reshape(_NPTS)
    color = colt.T
    af1 = sc_out[96:104, :].T
    af2 = sc_out[104:112, :].T
    return (sigma, color, af1, af2)
